# Initial kernel scaffold; baseline (speedup 1.0000x reference)
import jax
import jax.numpy as jnp
from jax.experimental import pallas as pl


def kernel(s_xyz, xyz, s_points, nsample):
    raise NotImplementedError("write your pallas kernel here")


# speedup vs baseline: 3.6368x; 3.6368x over previous
import functools

import jax
import jax.numpy as jnp
from jax import lax
from jax.experimental import pallas as pl
from jax.experimental.pallas import tpu as pltpu
from jax.experimental.pallas import tpu_sc as plsc

B, N, S, C, D = 2, 8192, 4096, 3, 128
K = 16
SB = 256


def _topk_tc_body(xyz_ref, sxyz_ref, idx_ref, gxyz_ref):
    b = pl.program_id(0)
    q = xyz_ref[0]
    s = sxyz_ref[0]

    sT = jnp.concatenate(
        [s[:, 0].reshape(1, N), s[:, 1].reshape(1, N), s[:, 2].reshape(1, N)],
        axis=0)
    dot = lax.dot_general(
        q, sT, (((1,), (0,)), ((), ())),
        preferred_element_type=jnp.float32)
    qn = jnp.sum(q * q, axis=1, keepdims=True)
    sn = jnp.sum(s * s, axis=1).reshape(1, N)
    dist = -2.0 * dot
    dist = dist + qn
    dist = dist + sn
    neg = -dist

    iota = lax.broadcasted_iota(jnp.int32, (SB, N), 1)
    kcol16 = lax.broadcasted_iota(jnp.int32, (SB, K), 1)
    kcol48 = lax.broadcasted_iota(jnp.int32, (SB, K * C), 1) // C

    def step(k, carry):
        neg, idx_acc, gxyz_acc = carry
        m = jnp.max(neg, axis=1, keepdims=True)
        is_m = neg == m
        idxk = jnp.min(jnp.where(is_m, iota, N), axis=1)
        onehot = iota == idxk[:, None]
        sel = lax.dot_general(
            onehot.astype(jnp.float32), s,
            (((1,), (0,)), ((), ())),
            precision=lax.Precision.HIGHEST,
            preferred_element_type=jnp.float32)
        gk = jnp.tile(sel - q, (1, K))
        idx_acc = jnp.where(kcol16 == k, (idxk + b * N)[:, None], idx_acc)
        gxyz_acc = jnp.where(kcol48 == k, gk, gxyz_acc)
        neg = jnp.where(onehot, -jnp.inf, neg)
        return neg, idx_acc, gxyz_acc

    _, idx_acc, gxyz_acc = lax.fori_loop(
        0, K, step,
        (neg,
         jnp.zeros((SB, K), jnp.int32),
         jnp.zeros((SB, K * C), jnp.float32)))
    idx_ref[0] = idx_acc
    gxyz_ref[0] = gxyz_acc


def _topk_tc(xyz, s_xyz):
    grid = (B, S // SB)
    return pl.pallas_call(
        _topk_tc_body,
        grid=grid,
        in_specs=[
            pl.BlockSpec((1, SB, C), lambda b, j: (b, j, 0)),
            pl.BlockSpec((1, N, C), lambda b, j: (b, 0, 0)),
        ],
        out_specs=[
            pl.BlockSpec((1, SB, K), lambda b, j: (b, j, 0)),
            pl.BlockSpec((1, SB, K * C), lambda b, j: (b, j, 0)),
        ],
        out_shape=[
            jax.ShapeDtypeStruct((B, S, K), jnp.int32),
            jax.ShapeDtypeStruct((B, S, K * C), jnp.float32),
        ],
    )(xyz, s_xyz)


_NC, _NS = 2, 16
_NW = _NC * _NS
_R = B * S * K
_RW = _R // _NW
_CHUNK = 128


def _gather_sc_body(table_hbm, idx_hbm, out_hbm, idx_v, rows_v, sem):
    wid = lax.axis_index("s") * _NC + lax.axis_index("c")
    base = wid * _RW
    pltpu.sync_copy(idx_hbm.at[pl.ds(base, _RW)], idx_v)

    def chunk(c, carry):
        off = c * _CHUNK
        pltpu.async_copy(
            table_hbm.at[idx_v.at[pl.ds(off, _CHUNK)]], rows_v, sem).wait()
        pltpu.sync_copy(rows_v, out_hbm.at[pl.ds(base + off, _CHUNK)])
        return carry

    lax.fori_loop(0, _RW // _CHUNK, chunk, 0)


@functools.lru_cache(maxsize=None)
def _gather_sc():
    return pl.kernel(
        _gather_sc_body,
        mesh=plsc.VectorSubcoreMesh(core_axis_name="c", subcore_axis_name="s"),
        out_type=jax.ShapeDtypeStruct((_R, D), jnp.float32),
        scratch_types=[
            pltpu.VMEM((_RW,), jnp.int32),
            pltpu.VMEM((_CHUNK, D), jnp.float32),
            pltpu.SemaphoreType.DMA,
        ],
    )


def kernel(s_xyz, xyz, s_points, nsample):
    idx_flat, gxyz = _topk_tc(xyz, s_xyz)
    grouped = _gather_sc()(s_points.reshape(B * N, D), idx_flat.reshape(_R))
    grouped_xyz_norm = gxyz.reshape(B, S, K, C)
    new_points = jnp.concatenate(
        [grouped_xyz_norm, grouped.reshape(B, S, K, D)], axis=-1)
    return new_points, grouped_xyz_norm


# speedup vs baseline: 8.0098x; 2.2024x over previous
import functools

import jax
import jax.numpy as jnp
from jax import lax
from jax.experimental import pallas as pl
from jax.experimental.pallas import tpu as pltpu
from jax.experimental.pallas import tpu_sc as plsc

B, N, S, C, D = 2, 8192, 4096, 3, 128
K = 16
SB = 256
LO = 128
HI = N // LO


def _topk_tc_body(xyz_ref, sxyz_ref, spack_ref, idx_ref, gxyz_ref):
    b = pl.program_id(0)
    q = xyz_ref[0]
    s = sxyz_ref[0]
    spack = spack_ref[0]

    sT = jnp.concatenate(
        [s[:, 0].reshape(1, N), s[:, 1].reshape(1, N), s[:, 2].reshape(1, N)],
        axis=0)
    dot = lax.dot_general(
        q, sT, (((1,), (0,)), ((), ())),
        preferred_element_type=jnp.float32)
    qn = jnp.sum(q * q, axis=1, keepdims=True)
    sn = jnp.sum(s * s, axis=1).reshape(1, N)
    dist = -2.0 * dot
    dist = dist + qn
    dist = dist + sn
    neg = -dist

    iota = lax.broadcasted_iota(jnp.int32, (SB, N), 1)
    kcol16 = lax.broadcasted_iota(jnp.int32, (SB, K), 1)
    kcol48 = lax.broadcasted_iota(jnp.int32, (SB, K * C), 1) // C
    iota_lo = lax.broadcasted_iota(jnp.int32, (SB, LO), 1)
    lane_hc = lax.broadcasted_iota(jnp.int32, (SB, HI * C), 1)

    def step(k, carry):
        m_prev, i_prev, idx_acc, gxyz_acc = carry
        cond = (neg < m_prev[:, None]) | (
            (neg == m_prev[:, None]) & (iota > i_prev[:, None]))
        m = jnp.max(jnp.where(cond, neg, -jnp.inf), axis=1)
        i = jnp.min(jnp.where(cond & (neg == m[:, None]), iota, N),
                    axis=1)
        ilo = lax.rem(i, LO)
        ihi = lax.div(i, LO)
        onehot_lo = (iota_lo == ilo[:, None]).astype(jnp.float32)
        tmp = lax.dot_general(
            onehot_lo, spack, (((1,), (0,)), ((), ())),
            precision=lax.Precision.HIGHEST,
            preferred_element_type=jnp.float32)
        h3 = ihi * C
        selx = jnp.sum(jnp.where(lane_hc == h3[:, None], tmp, 0.0), axis=1)
        sely = jnp.sum(jnp.where(lane_hc == (h3 + 1)[:, None], tmp, 0.0), axis=1)
        selz = jnp.sum(jnp.where(lane_hc == (h3 + 2)[:, None], tmp, 0.0), axis=1)
        gsel = jnp.concatenate(
            [selx[:, None], sely[:, None], selz[:, None]], axis=1) - q
        idx_acc = jnp.where(kcol16 == k, (i + b * N)[:, None], idx_acc)
        gxyz_acc = jnp.where(kcol48 == k, jnp.tile(gsel, (1, K)), gxyz_acc)
        return m, i, idx_acc, gxyz_acc

    _, _, idx_acc, gxyz_acc = lax.fori_loop(
        0, K, step,
        (jnp.full((SB,), jnp.inf, jnp.float32),
         jnp.full((SB,), -1, jnp.int32),
         jnp.zeros((SB, K), jnp.int32),
         jnp.zeros((SB, K * C), jnp.float32)))
    idx_ref[0] = idx_acc
    gxyz_ref[0] = gxyz_acc


def _topk_tc(xyz, s_xyz, s_pack):
    grid = (B, S // SB)
    return pl.pallas_call(
        _topk_tc_body,
        grid=grid,
        in_specs=[
            pl.BlockSpec((1, SB, C), lambda b, j: (b, j, 0)),
            pl.BlockSpec((1, N, C), lambda b, j: (b, 0, 0)),
            pl.BlockSpec((1, LO, HI * C), lambda b, j: (b, 0, 0)),
        ],
        out_specs=[
            pl.BlockSpec((1, SB, K), lambda b, j: (b, j, 0)),
            pl.BlockSpec((1, SB, K * C), lambda b, j: (b, j, 0)),
        ],
        out_shape=[
            jax.ShapeDtypeStruct((B, S, K), jnp.int32),
            jax.ShapeDtypeStruct((B, S, K * C), jnp.float32),
        ],
    )(xyz, s_xyz, s_pack)


_NC, _NS = 2, 16
_NW = _NC * _NS
_R = B * S * K
_RW = _R // _NW
_CHUNK = 128


def _gather_sc_body(table_hbm, idx_hbm, out_hbm, idx_v, rows_v, sem):
    wid = lax.axis_index("s") * _NC + lax.axis_index("c")
    base = wid * _RW
    pltpu.sync_copy(idx_hbm.at[pl.ds(base, _RW)], idx_v)

    def chunk(c, carry):
        off = c * _CHUNK
        pltpu.async_copy(
            table_hbm.at[idx_v.at[pl.ds(off, _CHUNK)]], rows_v, sem).wait()
        pltpu.sync_copy(rows_v, out_hbm.at[pl.ds(base + off, _CHUNK)])
        return carry

    lax.fori_loop(0, _RW // _CHUNK, chunk, 0)


@functools.lru_cache(maxsize=None)
def _gather_sc():
    return pl.kernel(
        _gather_sc_body,
        mesh=plsc.VectorSubcoreMesh(core_axis_name="c", subcore_axis_name="s"),
        out_type=jax.ShapeDtypeStruct((_R, D), jnp.float32),
        scratch_types=[
            pltpu.VMEM((_RW,), jnp.int32),
            pltpu.VMEM((_CHUNK, D), jnp.float32),
            pltpu.SemaphoreType.DMA,
        ],
    )


def kernel(s_xyz, xyz, s_points, nsample):
    s_pack = jnp.transpose(
        s_xyz.reshape(B, HI, LO, C), (0, 2, 1, 3)).reshape(B, LO, HI * C)
    idx_flat, gxyz = _topk_tc(xyz, s_xyz, s_pack)
    grouped = _gather_sc()(s_points.reshape(B * N, D), idx_flat.reshape(_R))
    grouped_xyz_norm = gxyz.reshape(B, S, K, C)
    new_points = jnp.concatenate(
        [grouped_xyz_norm, grouped.reshape(B, S, K, D)], axis=-1)
    return new_points, grouped_xyz_norm


# speedup vs baseline: 8.1147x; 1.0131x over previous
import functools

import jax
import jax.numpy as jnp
from jax import lax
from jax.experimental import pallas as pl
from jax.experimental.pallas import tpu as pltpu
from jax.experimental.pallas import tpu_sc as plsc

B, N, S, C, D = 2, 8192, 4096, 3, 128
K = 16
SB = 256
LO = 128
HI = N // LO


def _topk_tc_body(xyz_ref, sxyz_ref, sxyzT_ref, spack_ref, idx_ref, gxyz_ref):
    b = pl.program_id(0)
    q = xyz_ref[0]
    s = sxyz_ref[0]
    sT = sxyzT_ref[0]
    spack = spack_ref[0]

    dot = lax.dot_general(
        q, sT, (((1,), (0,)), ((), ())),
        preferred_element_type=jnp.float32)
    qn = jnp.sum(q * q, axis=1, keepdims=True)
    sn = jnp.sum(s * s, axis=1).reshape(1, N)
    dist = -2.0 * dot
    dist = dist + qn
    dist = dist + sn
    neg0 = -dist

    iota = lax.broadcasted_iota(jnp.int32, (SB, N), 1)
    kcol16 = lax.broadcasted_iota(jnp.int32, (SB, K), 1)
    kcol48 = lax.broadcasted_iota(jnp.int32, (SB, K * C), 1) // C
    iota_lo = lax.broadcasted_iota(jnp.int32, (SB, LO), 1)
    lane_hc = lax.broadcasted_iota(jnp.int32, (SB, HI * C), 1)

    def step(k, carry):
        neg, idx_acc, gxyz_acc = carry
        m = jnp.max(neg, axis=1)
        i = jnp.min(jnp.where(neg == m[:, None], iota, N), axis=1)
        neg = jnp.where(iota == i[:, None], -jnp.inf, neg)
        ilo = lax.rem(i, LO)
        ihi = lax.div(i, LO)
        onehot_lo = (iota_lo == ilo[:, None]).astype(jnp.float32)
        tmp = lax.dot_general(
            onehot_lo, spack, (((1,), (0,)), ((), ())),
            precision=lax.Precision.HIGHEST,
            preferred_element_type=jnp.float32)
        h3 = ihi * C
        selx = jnp.sum(jnp.where(lane_hc == h3[:, None], tmp, 0.0), axis=1)
        sely = jnp.sum(jnp.where(lane_hc == (h3 + 1)[:, None], tmp, 0.0), axis=1)
        selz = jnp.sum(jnp.where(lane_hc == (h3 + 2)[:, None], tmp, 0.0), axis=1)
        gsel = jnp.concatenate(
            [selx[:, None], sely[:, None], selz[:, None]], axis=1) - q
        idx_acc = jnp.where(kcol16 == k, (i + b * N)[:, None], idx_acc)
        gxyz_acc = jnp.where(kcol48 == k, jnp.tile(gsel, (1, K)), gxyz_acc)
        return neg, idx_acc, gxyz_acc

    _, idx_acc, gxyz_acc = lax.fori_loop(
        0, K, step,
        (neg0,
         jnp.zeros((SB, K), jnp.int32),
         jnp.zeros((SB, K * C), jnp.float32)))
    idx_ref[0] = idx_acc
    gxyz_ref[0] = gxyz_acc


def _topk_tc(xyz, s_xyz, s_xyzT, s_pack):
    grid = (B, S // SB)
    return pl.pallas_call(
        _topk_tc_body,
        grid=grid,
        in_specs=[
            pl.BlockSpec((1, SB, C), lambda b, j: (b, j, 0)),
            pl.BlockSpec((1, N, C), lambda b, j: (b, 0, 0)),
            pl.BlockSpec((1, C, N), lambda b, j: (b, 0, 0)),
            pl.BlockSpec((1, LO, HI * C), lambda b, j: (b, 0, 0)),
        ],
        out_specs=[
            pl.BlockSpec((1, SB, K), lambda b, j: (b, j, 0)),
            pl.BlockSpec((1, SB, K * C), lambda b, j: (b, j, 0)),
        ],
        out_shape=[
            jax.ShapeDtypeStruct((B, S, K), jnp.int32),
            jax.ShapeDtypeStruct((B, S, K * C), jnp.float32),
        ],
    )(xyz, s_xyz, s_xyzT, s_pack)


_NC, _NS = 2, 16
_NW = _NC * _NS
_R = B * S * K
_RW = _R // _NW
_CHUNK = 128


def _gather_sc_body(table_hbm, idx_hbm, out_hbm, idx_v, rows_v, sem):
    wid = lax.axis_index("s") * _NC + lax.axis_index("c")
    base = wid * _RW
    pltpu.sync_copy(idx_hbm.at[pl.ds(base, _RW)], idx_v)

    def chunk(c, carry):
        off = c * _CHUNK
        pltpu.async_copy(
            table_hbm.at[idx_v.at[pl.ds(off, _CHUNK)]], rows_v, sem).wait()
        pltpu.sync_copy(rows_v, out_hbm.at[pl.ds(base + off, _CHUNK)])
        return carry

    lax.fori_loop(0, _RW // _CHUNK, chunk, 0)


@functools.lru_cache(maxsize=None)
def _gather_sc():
    return pl.kernel(
        _gather_sc_body,
        mesh=plsc.VectorSubcoreMesh(core_axis_name="c", subcore_axis_name="s"),
        out_type=jax.ShapeDtypeStruct((_R, D), jnp.float32),
        scratch_types=[
            pltpu.VMEM((_RW,), jnp.int32),
            pltpu.VMEM((_CHUNK, D), jnp.float32),
            pltpu.SemaphoreType.DMA,
        ],
    )


def kernel(s_xyz, xyz, s_points, nsample):
    s_pack = jnp.transpose(
        s_xyz.reshape(B, HI, LO, C), (0, 2, 1, 3)).reshape(B, LO, HI * C)
    s_xyzT = jnp.swapaxes(s_xyz, 1, 2)
    idx_flat, gxyz = _topk_tc(xyz, s_xyz, s_xyzT, s_pack)
    grouped = _gather_sc()(s_points.reshape(B * N, D), idx_flat.reshape(_R))
    grouped_xyz_norm = gxyz.reshape(B, S, K, C)
    new_points = jnp.concatenate(
        [grouped_xyz_norm, grouped.reshape(B, S, K, D)], axis=-1)
    return new_points, grouped_xyz_norm


# speedup vs baseline: 11.1076x; 1.3688x over previous
import functools

import jax
import jax.numpy as jnp
from jax import lax
from jax.experimental import pallas as pl
from jax.experimental.pallas import tpu as pltpu
from jax.experimental.pallas import tpu_sc as plsc

B, N, S, C, D = 2, 8192, 4096, 3, 128
K = 16
SB = 256
AW = 256
OW = C + D
LANES = 16


def _topk_tc_body(xyz_ref, sxyz_ref, sxyzT_ref, idx_ref):
    b = pl.program_id(0)
    q = xyz_ref[0]
    s = sxyz_ref[0]
    sT = sxyzT_ref[0]

    dot = lax.dot_general(
        q, sT, (((1,), (0,)), ((), ())),
        preferred_element_type=jnp.float32)
    qn = jnp.sum(q * q, axis=1, keepdims=True)
    sn = jnp.sum(s * s, axis=1).reshape(1, N)
    dist = -2.0 * dot
    dist = dist + qn
    dist = dist + sn
    neg0 = -dist

    iota_f = lax.broadcasted_iota(
        jnp.int32, (SB, N), 1).astype(jnp.float32)
    kcol16 = lax.broadcasted_iota(jnp.int32, (SB, K), 1)
    big = jnp.float32(N)

    def step(k, carry):
        i_prev, neg, idx_acc = carry
        neg = jnp.where(iota_f == i_prev[:, None], -jnp.inf, neg)
        m = jnp.max(neg, axis=1)
        i_f = jnp.min(jnp.where(neg == m[:, None], iota_f, big),
                      axis=1)
        i = i_f.astype(jnp.int32)
        idx_acc = jnp.where(kcol16 == k, (i + b * N)[:, None], idx_acc)
        return i_f, neg, idx_acc

    _, _, idx_acc = lax.fori_loop(
        0, K, step,
        (jnp.full((SB,), -1.0, jnp.float32),
         neg0,
         jnp.zeros((SB, K), jnp.int32)))
    idx_ref[0] = idx_acc


def _topk_tc(xyz, s_xyz, s_xyzT):
    grid = (B, S // SB)
    return pl.pallas_call(
        _topk_tc_body,
        grid=grid,
        in_specs=[
            pl.BlockSpec((1, SB, C), lambda b, j: (b, j, 0)),
            pl.BlockSpec((1, N, C), lambda b, j: (b, 0, 0)),
            pl.BlockSpec((1, C, N), lambda b, j: (b, 0, 0)),
        ],
        out_specs=pl.BlockSpec((1, SB, K), lambda b, j: (b, j, 0)),
        out_shape=jax.ShapeDtypeStruct((B, S, K), jnp.int32),
    )(xyz, s_xyz, s_xyzT)


_NC, _NS = 2, 16
_NW = _NC * _NS
_R = B * S * K
_RW = _R // _NW
_QW = _RW // K
_CHUNK = 128
_QC = _CHUNK // K


def _gather_sc_body(table_hbm, idx_hbm, qp_hbm, out_hbm,
                    idx_v, rows_v, q_v, sem):
    wid = lax.axis_index("s") * _NC + lax.axis_index("c")
    base = wid * _RW
    pltpu.sync_copy(idx_hbm.at[pl.ds(base, _RW)], idx_v)
    pltpu.sync_copy(qp_hbm.at[pl.ds(wid * _QW, _QW)], q_v)

    def chunk(c, carry):
        off = c * _CHUNK
        pltpu.async_copy(
            table_hbm.at[idx_v.at[pl.ds(off, _CHUNK)]], rows_v, sem).wait()
        qbase = c * _QC
        for r in range(_CHUNK):
            rows_v[r, pl.ds(0, LANES)] = (
                rows_v[r, pl.ds(0, LANES)] - q_v[qbase + r // K])
        pltpu.sync_copy(rows_v, out_hbm.at[pl.ds(base + off, _CHUNK)])
        return carry

    lax.fori_loop(0, _RW // _CHUNK, chunk, 0)


@functools.lru_cache(maxsize=None)
def _gather_sc():
    return pl.kernel(
        _gather_sc_body,
        mesh=plsc.VectorSubcoreMesh(core_axis_name="c", subcore_axis_name="s"),
        out_type=jax.ShapeDtypeStruct((_R, AW), jnp.float32),
        scratch_types=[
            pltpu.VMEM((_RW,), jnp.int32),
            pltpu.VMEM((_CHUNK, AW), jnp.float32),
            pltpu.VMEM((_QW, LANES), jnp.float32),
            pltpu.SemaphoreType.DMA,
        ],
    )


def kernel(s_xyz, xyz, s_points, nsample):
    s_xyzT = jnp.swapaxes(s_xyz, 1, 2)
    idx_flat = _topk_tc(xyz, s_xyz, s_xyzT)
    aug = jnp.concatenate(
        [s_xyz.reshape(B * N, C), s_points.reshape(B * N, D),
         jnp.zeros((B * N, AW - OW), jnp.float32)], axis=1)
    q_pad = jnp.pad(xyz.reshape(B * S, C), ((0, 0), (0, LANES - C)))
    rows = _gather_sc()(aug, idx_flat.reshape(_R), q_pad)
    new_points = rows[:, :OW].reshape(B, S, K, OW)
    grouped_xyz_norm = new_points[..., :C]
    return new_points, grouped_xyz_norm
